# baseline (speedup 1.0000x reference)
import functools

import jax
import jax.numpy as jnp
from jax import lax
from jax.experimental import pallas as pl
from jax.experimental.pallas import tpu as pltpu
from jax.experimental.pallas import tpu_sc as plsc

D = 64
SCALE = 8.0
LANES = 16
NBUF = 4


def _build_sc_kernel(R: int, C: int, V: int):
    info = plsc.get_sparse_core_info()
    NC, NS = info.num_cores, info.num_subcores
    NW = NC * NS
    rows_w = R // NW
    S = rows_w // NBUF
    assert R % NW == 0 and rows_w % NBUF == 0 and S >= 3
    splits = []
    off = 0
    while off < C:
        w = min(128, C - off)
        splits.append((off, w))
        off += w
    assert sum(w for _, w in splits) == C and all(o % 8 == 0 for o, _ in splits)

    mesh = plsc.VectorSubcoreMesh(core_axis_name="c", subcore_axis_name="s")

    @functools.partial(
        pl.kernel,
        mesh=mesh,
        out_type=jax.ShapeDtypeStruct((R, C, D), jnp.float32),
        scratch_types=[
            pltpu.VMEM((rows_w, C), jnp.int32),
            *[pltpu.VMEM((C, D), jnp.float32) for _ in range(NBUF)],
            *[pltpu.SemaphoreType.DMA for _ in range(2 * NBUF)],
        ],
        compiler_params=pltpu.CompilerParams(use_tc_tiling_on_sc=False),
    )
    def k(x_hbm, table_hbm, out_hbm, idx_v, *bufs_and_sems):
        rows = bufs_and_sems[:NBUF]
        gsem = bufs_and_sems[NBUF:2 * NBUF]
        osem = bufs_and_sems[2 * NBUF:]

        wid = lax.axis_index("s") * NC + lax.axis_index("c")
        base = wid * rows_w

        pltpu.sync_copy(x_hbm.at[pl.ds(base, rows_w)], idx_v)

        def fire_gather(i, b):
            for off, w in splits:
                pltpu.async_copy(
                    table_hbm.at[idx_v.at[i, pl.ds(off, w)]],
                    rows[b].at[pl.ds(off, w)],
                    gsem[b],
                )

        def drain_gather(b):
            pltpu.make_async_copy(
                table_hbm.at[pl.ds(0, C)], rows[b], gsem[b]).wait()

        def wait_out(b):
            pltpu.make_async_copy(rows[b], out_hbm.at[0], osem[b]).wait()

        def scale_buf(b):
            def body(i, carry):
                for j in range(D // LANES):
                    sl = (i, pl.ds(j * LANES, LANES))
                    rows[b][sl] = rows[b][sl] * SCALE
                return carry
            lax.fori_loop(0, C, body, 0)

        def step(i, b, first, last):
            drain_gather(b)
            scale_buf(b)
            pltpu.async_copy(rows[b], out_hbm.at[base + i], osem[b])
            nb = (b + 2) % NBUF
            if not last:
                if not first:
                    wait_out(nb)
                fire_gather(i + 2, nb)
            elif not first:
                wait_out(nb)

        fire_gather(0, 0)
        fire_gather(1, 1)

        for b in range(NBUF):
            step(b, b, first=(b < 2), last=False)

        def rev(s, carry):
            for b in range(NBUF):
                step(s * NBUF + b, b, first=False, last=False)
            return carry
        lax.fori_loop(1, S - 1, rev, 0)

        for b in range(NBUF):
            i = (S - 1) * NBUF + b
            step(i, b, first=False, last=(b >= 2))
        wait_out(2)
        wait_out(3)

    return k


def kernel(x, table):
    R, C = x.shape
    V = table.shape[0]
    return _build_sc_kernel(R, C, V)(x.astype(jnp.int32), table)
